# baseline (speedup 1.0000x reference)
import functools

import jax
import jax.numpy as jnp
from jax import lax
from jax.experimental import pallas as pl
from jax.experimental.pallas import tpu as pltpu
from jax.experimental.pallas import tpu_sc as plsc

VOCAB = 1000000
D = 64
B_TOTAL = 4096 * 50

_info = plsc.get_sparse_core_info()
NC = _info.num_cores
NS = _info.num_subcores
NW = NC * NS
B_PER_W = B_TOTAL // NW
CHUNK = 128
N_CHUNKS = B_PER_W // CHUNK


def _make_gather():
    mesh = plsc.VectorSubcoreMesh(core_axis_name="c", subcore_axis_name="s")

    @functools.partial(
        pl.kernel,
        mesh=mesh,
        out_type=jax.ShapeDtypeStruct((B_TOTAL, D), jnp.float32),
        scratch_types=[
            pltpu.VMEM((B_PER_W,), jnp.int32),
            pltpu.VMEM((CHUNK, D), jnp.float32),
            pltpu.SemaphoreType.DMA,
        ],
        compiler_params=pltpu.CompilerParams(use_tc_tiling_on_sc=False),
    )
    def gather_kernel(table_hbm, ids_hbm, out_hbm, idx_v, rows_v, sem):
        wid = lax.axis_index("s") * NC + lax.axis_index("c")
        base = wid * B_PER_W
        pltpu.sync_copy(ids_hbm.at[pl.ds(base, B_PER_W)], idx_v)

        def step(j, carry):
            off = pl.multiple_of(j * CHUNK, CHUNK)
            pltpu.async_copy(
                table_hbm.at[idx_v.at[pl.ds(off, CHUNK)]], rows_v, sem
            ).wait()
            pltpu.sync_copy(rows_v, out_hbm.at[pl.ds(base + off, CHUNK)])
            return carry

        lax.fori_loop(0, N_CHUNKS, step, 0)

    return gather_kernel


_gather = _make_gather()


@jax.jit
def kernel(input_ids, weight):
    ids_flat = input_ids.astype(jnp.int32).reshape(B_TOTAL)
    out = _gather(weight, ids_flat)
    return out.reshape(input_ids.shape[0], input_ids.shape[1], D)


# speedup vs baseline: 1.0418x; 1.0418x over previous
import functools

import jax
import jax.numpy as jnp
from jax import lax
from jax.experimental import pallas as pl
from jax.experimental.pallas import tpu as pltpu
from jax.experimental.pallas import tpu_sc as plsc

VOCAB = 1000000
D = 64
B_TOTAL = 4096 * 50

_info = plsc.get_sparse_core_info()
NC = _info.num_cores
NS = _info.num_subcores
NW = NC * NS
B_PER_W = B_TOTAL // NW
CHUNK = 128
GC = 5
GROUP = GC * CHUNK
NG = B_PER_W // GROUP
NB = 2


def _make_gather():
    mesh = plsc.VectorSubcoreMesh(core_axis_name="c", subcore_axis_name="s")

    @functools.partial(
        pl.kernel,
        mesh=mesh,
        out_type=jax.ShapeDtypeStruct((B_TOTAL, D), jnp.float32),
        scratch_types=[
            pltpu.VMEM((B_PER_W,), jnp.int32),
            pltpu.VMEM((GROUP, D), jnp.float32),
            pltpu.VMEM((GROUP, D), jnp.float32),
            pltpu.SemaphoreType.DMA,
            pltpu.SemaphoreType.DMA,
            pltpu.SemaphoreType.DMA,
            pltpu.SemaphoreType.DMA,
        ],
        compiler_params=pltpu.CompilerParams(use_tc_tiling_on_sc=False),
    )
    def gather_kernel(table_hbm, ids_hbm, out_hbm,
                      idx_v, rows_a, rows_b, sg_a, sg_b, ss_a, ss_b):
        wid = lax.axis_index("s") * NC + lax.axis_index("c")
        base = wid * B_PER_W
        pltpu.sync_copy(ids_hbm.at[pl.ds(base, B_PER_W)], idx_v)

        rows = (rows_a, rows_b)
        sg = (sg_a, sg_b)
        ss = (ss_a, ss_b)

        @pl.loop(0, NG, step=NB)
        def _group(go):
            for b in range(NB):
                g = go + b
                buf = rows[b]
                @pl.when(go >= NB)
                def _():
                    pltpu.make_async_copy(
                        buf, out_hbm.at[pl.ds(base, GROUP)], ss[b]
                    ).wait()

                goff = pl.multiple_of(g * GROUP, GROUP)
                descs = []
                for c in range(GC):
                    d = pltpu.make_async_copy(
                        table_hbm.at[idx_v.at[pl.ds(goff + c * CHUNK, CHUNK)]],
                        buf.at[pl.ds(c * CHUNK, CHUNK)],
                        sg[b],
                    )
                    d.start()
                    descs.append(d)
                for d in descs:
                    d.wait()
                pltpu.make_async_copy(
                    buf, out_hbm.at[pl.ds(base + goff, GROUP)], ss[b]
                ).start()

        for b in range(NB):
            pltpu.make_async_copy(
                rows[b], out_hbm.at[pl.ds(base, GROUP)], ss[b]
            ).wait()

    return gather_kernel


_gather = _make_gather()


@jax.jit
def kernel(input_ids, weight):
    ids_flat = input_ids.astype(jnp.int32).reshape(B_TOTAL)
    out = _gather(weight, ids_flat)
    return out.reshape(input_ids.shape[0], input_ids.shape[1], D)


# speedup vs baseline: 1.5876x; 1.5239x over previous
import functools

import jax
import jax.numpy as jnp
from jax import lax
from jax.experimental import pallas as pl
from jax.experimental.pallas import tpu as pltpu
from jax.experimental.pallas import tpu_sc as plsc

VOCAB = 1000000
D = 64
B_TOTAL = 4096 * 50

PANEL = 2048
NPAN = (VOCAB + 2 * PANEL - 1) // (2 * PANEL)
VOCAB_V = NPAN * 2 * PANEL

_info = plsc.get_sparse_core_info()
NC = _info.num_cores
NS = _info.num_subcores
NW = NC * NS
B_PER_W = B_TOTAL // NW
CHUNK = 128
GC = 5
GROUP = GC * CHUNK
NG = B_PER_W // GROUP
NB = 2


def _retile_body(wt_ref, out_ref):
    x = wt_ref[...]
    out_ref[:, 0:D] = x[:, 0:PANEL].T
    out_ref[:, D:2 * D] = x[:, PANEL:2 * PANEL].T


_retile = pl.pallas_call(
    _retile_body,
    grid=(NPAN,),
    in_specs=[pl.BlockSpec((D, 2 * PANEL), lambda c: (0, c))],
    out_specs=pl.BlockSpec((PANEL, 2 * D), lambda c: (c, 0)),
    out_shape=jax.ShapeDtypeStruct((NPAN * PANEL, 2 * D), jnp.float32),
)


def _make_gather():
    mesh = plsc.VectorSubcoreMesh(core_axis_name="c", subcore_axis_name="s")

    @functools.partial(
        pl.kernel,
        mesh=mesh,
        out_type=jax.ShapeDtypeStruct((B_TOTAL, D), jnp.float32),
        scratch_types=[
            pltpu.VMEM((B_PER_W,), jnp.int32),
            pltpu.VMEM((GROUP, D), jnp.float32),
            pltpu.VMEM((GROUP, D), jnp.float32),
            pltpu.SemaphoreType.DMA,
            pltpu.SemaphoreType.DMA,
            pltpu.SemaphoreType.DMA,
            pltpu.SemaphoreType.DMA,
        ],
        compiler_params=pltpu.CompilerParams(use_tc_tiling_on_sc=False),
    )
    def gather_kernel(table_hbm, ids_hbm, out_hbm,
                      idx_v, rows_a, rows_b, sg_a, sg_b, ss_a, ss_b):
        wid = lax.axis_index("s") * NC + lax.axis_index("c")
        base = wid * B_PER_W
        pltpu.sync_copy(ids_hbm.at[pl.ds(base, B_PER_W)], idx_v)

        rows = (rows_a, rows_b)
        sg = (sg_a, sg_b)
        ss = (ss_a, ss_b)

        @pl.loop(0, NG, step=NB)
        def _group(go):
            for b in range(NB):
                g = go + b
                buf = rows[b]
                @pl.when(go >= NB)
                def _():
                    pltpu.make_async_copy(
                        buf, out_hbm.at[pl.ds(base, GROUP)], ss[b]
                    ).wait()

                goff = pl.multiple_of(g * GROUP, GROUP)
                descs = []
                for c in range(GC):
                    d = pltpu.make_async_copy(
                        table_hbm.at[idx_v.at[pl.ds(goff + c * CHUNK, CHUNK)]],
                        buf.at[pl.ds(c * CHUNK, CHUNK)],
                        sg[b],
                    )
                    d.start()
                    descs.append(d)
                for d in descs:
                    d.wait()
                pltpu.make_async_copy(
                    buf, out_hbm.at[pl.ds(base + goff, GROUP)], ss[b]
                ).start()

        for b in range(NB):
            pltpu.make_async_copy(
                rows[b], out_hbm.at[pl.ds(base, GROUP)], ss[b]
            ).wait()

    return gather_kernel


_gather = _make_gather()


@jax.jit
def kernel(input_ids, weight):
    r = input_ids.astype(jnp.int32)
    rr = (r & ~(2 * PANEL - 1)) | ((r & (PANEL - 1)) << 1) | ((r // PANEL) & 1)
    ids_flat = rr.reshape(B_TOTAL)
    table = _retile(weight.T).reshape(VOCAB_V, D)
    out = _gather(table, ids_flat)
    return out.reshape(input_ids.shape[0], input_ids.shape[1], D)


# speedup vs baseline: 1.6046x; 1.0107x over previous
import functools

import jax
import jax.numpy as jnp
from jax import lax
from jax.experimental import pallas as pl
from jax.experimental.pallas import tpu as pltpu
from jax.experimental.pallas import tpu_sc as plsc

VOCAB = 1000000
D = 64
B_TOTAL = 4096 * 50

PANEL = 2048
NPAN = (VOCAB + 2 * PANEL - 1) // (2 * PANEL)
VOCAB_V = NPAN * 2 * PANEL

_info = plsc.get_sparse_core_info()
NC = _info.num_cores
NS = _info.num_subcores
NW = NC * NS
B_PER_W = B_TOTAL // NW
CHUNK = 128
GC = 5
GROUP = GC * CHUNK
NG = B_PER_W // GROUP
NB = 2


def _retile_body(wt_ref, out_ref):
    x = wt_ref[...]
    out_ref[:, 0:D] = x[:, 0:PANEL].T
    out_ref[:, D:2 * D] = x[:, PANEL:2 * PANEL].T


_retile = pl.pallas_call(
    _retile_body,
    grid=(NPAN,),
    in_specs=[pl.BlockSpec((D, 2 * PANEL), lambda c: (0, c))],
    out_specs=pl.BlockSpec((PANEL, 2 * D), lambda c: (c, 0)),
    out_shape=jax.ShapeDtypeStruct((NPAN * PANEL, 2 * D), jnp.float32),
)

SEQ = 50
BATCH = 4096
HB = BATCH // 2


def _outt_body(g_ref, o_ref):
    x = g_ref[...]
    o_ref[0, :, 0:HB] = x[:, 0:D].T
    o_ref[0, :, HB:BATCH] = x[:, D:2 * D].T


_outt = pl.pallas_call(
    _outt_body,
    grid=(SEQ,),
    in_specs=[pl.BlockSpec((HB, 2 * D), lambda s: (s, 0))],
    out_specs=pl.BlockSpec((1, D, BATCH), lambda s: (s, 0, 0)),
    out_shape=jax.ShapeDtypeStruct((SEQ, D, BATCH), jnp.float32),
)


def _make_gather():
    mesh = plsc.VectorSubcoreMesh(core_axis_name="c", subcore_axis_name="s")

    @functools.partial(
        pl.kernel,
        mesh=mesh,
        out_type=jax.ShapeDtypeStruct((B_TOTAL, D), jnp.float32),
        scratch_types=[
            pltpu.VMEM((B_PER_W,), jnp.int32),
            pltpu.VMEM((GROUP, D), jnp.float32),
            pltpu.VMEM((GROUP, D), jnp.float32),
            pltpu.SemaphoreType.DMA,
            pltpu.SemaphoreType.DMA,
            pltpu.SemaphoreType.DMA,
            pltpu.SemaphoreType.DMA,
        ],
        compiler_params=pltpu.CompilerParams(use_tc_tiling_on_sc=False),
    )
    def gather_kernel(table_hbm, ids_hbm, out_hbm,
                      idx_v, rows_a, rows_b, sg_a, sg_b, ss_a, ss_b):
        wid = lax.axis_index("s") * NC + lax.axis_index("c")
        base = wid * B_PER_W
        pltpu.sync_copy(ids_hbm.at[pl.ds(base, B_PER_W)], idx_v)

        rows = (rows_a, rows_b)
        sg = (sg_a, sg_b)
        ss = (ss_a, ss_b)

        @pl.loop(0, NG, step=NB)
        def _group(go):
            for b in range(NB):
                g = go + b
                buf = rows[b]
                @pl.when(go >= NB)
                def _():
                    pltpu.make_async_copy(
                        buf, out_hbm.at[pl.ds(base, GROUP)], ss[b]
                    ).wait()

                goff = pl.multiple_of(g * GROUP, GROUP)
                descs = []
                for c in range(GC):
                    d = pltpu.make_async_copy(
                        table_hbm.at[idx_v.at[pl.ds(goff + c * CHUNK, CHUNK)]],
                        buf.at[pl.ds(c * CHUNK, CHUNK)],
                        sg[b],
                    )
                    d.start()
                    descs.append(d)
                for d in descs:
                    d.wait()
                pltpu.make_async_copy(
                    buf, out_hbm.at[pl.ds(base + goff, GROUP)], ss[b]
                ).start()

        for b in range(NB):
            pltpu.make_async_copy(
                rows[b], out_hbm.at[pl.ds(base, GROUP)], ss[b]
            ).wait()

    return gather_kernel


_gather = _make_gather()


@jax.jit
def kernel(input_ids, weight):
    r = input_ids.astype(jnp.int32)
    rr = (r & ~(2 * PANEL - 1)) | ((r & (PANEL - 1)) << 1) | ((r // PANEL) & 1)
    ids_flat = (
        jnp.transpose(rr.T.reshape(SEQ, 2, HB), (0, 2, 1)).reshape(B_TOTAL)
    )
    table = _retile(weight.T).reshape(VOCAB_V, D)
    g = _gather(table, ids_flat)
    o_p = _outt(g.reshape(B_TOTAL // 2, 2 * D))
    return jnp.transpose(o_p, (2, 0, 1))


# speedup vs baseline: 2.0674x; 1.2884x over previous
import functools

import jax
import jax.numpy as jnp
from jax import lax
from jax.experimental import pallas as pl
from jax.experimental.pallas import tpu as pltpu
from jax.experimental.pallas import tpu_sc as plsc

VOCAB = 1000000
D = 64
B_TOTAL = 4096 * 50

PANEL = 16384
NPAN = (VOCAB + 2 * PANEL - 1) // (2 * PANEL)
VOCAB_V = NPAN * 2 * PANEL

_info = plsc.get_sparse_core_info()
NC = _info.num_cores
NS = _info.num_subcores
NW = NC * NS
B_PER_W = B_TOTAL // NW
CHUNK = 128
GC = 5
GROUP = GC * CHUNK
NG = B_PER_W // GROUP
NB = 2


def _retile_body(wt_ref, out_ref):
    x = wt_ref[...]
    out_ref[:, 0:D] = x[:, 0:PANEL].T
    out_ref[:, D:2 * D] = x[:, PANEL:2 * PANEL].T


_retile = pl.pallas_call(
    _retile_body,
    grid=(NPAN,),
    in_specs=[pl.BlockSpec((D, 2 * PANEL), lambda c: (0, c))],
    out_specs=pl.BlockSpec((PANEL, 2 * D), lambda c: (c, 0)),
    out_shape=jax.ShapeDtypeStruct((NPAN * PANEL, 2 * D), jnp.float32),
)

SEQ = 50
BATCH = 4096
HB = BATCH // 2


def _outt_body(g_ref, o_ref):
    x = g_ref[...]
    o_ref[0, :, 0:HB] = x[:, 0:D].T
    o_ref[0, :, HB:BATCH] = x[:, D:2 * D].T


_outt = pl.pallas_call(
    _outt_body,
    grid=(SEQ,),
    in_specs=[pl.BlockSpec((HB, 2 * D), lambda s: (s, 0))],
    out_specs=pl.BlockSpec((1, D, BATCH), lambda s: (s, 0, 0)),
    out_shape=jax.ShapeDtypeStruct((SEQ, D, BATCH), jnp.float32),
)


def _make_gather():
    mesh = plsc.VectorSubcoreMesh(core_axis_name="c", subcore_axis_name="s")

    @functools.partial(
        pl.kernel,
        mesh=mesh,
        out_type=jax.ShapeDtypeStruct((B_TOTAL, D), jnp.float32),
        scratch_types=[
            pltpu.VMEM((B_PER_W,), jnp.int32),
            pltpu.VMEM((GROUP, D), jnp.float32),
            pltpu.VMEM((GROUP, D), jnp.float32),
            pltpu.SemaphoreType.DMA,
            pltpu.SemaphoreType.DMA,
            pltpu.SemaphoreType.DMA,
            pltpu.SemaphoreType.DMA,
        ],
        compiler_params=pltpu.CompilerParams(use_tc_tiling_on_sc=False),
    )
    def gather_kernel(table_hbm, ids_hbm, out_hbm,
                      idx_v, rows_a, rows_b, sg_a, sg_b, ss_a, ss_b):
        wid = lax.axis_index("s") * NC + lax.axis_index("c")
        base = wid * B_PER_W
        pltpu.sync_copy(ids_hbm.at[pl.ds(base, B_PER_W)], idx_v)

        rows = (rows_a, rows_b)
        sg = (sg_a, sg_b)
        ss = (ss_a, ss_b)

        @pl.loop(0, NG, step=NB)
        def _group(go):
            for b in range(NB):
                g = go + b
                buf = rows[b]
                @pl.when(go >= NB)
                def _():
                    pltpu.make_async_copy(
                        buf, out_hbm.at[pl.ds(base, GROUP)], ss[b]
                    ).wait()

                goff = pl.multiple_of(g * GROUP, GROUP)
                descs = []
                for c in range(GC):
                    d = pltpu.make_async_copy(
                        table_hbm.at[idx_v.at[pl.ds(goff + c * CHUNK, CHUNK)]],
                        buf.at[pl.ds(c * CHUNK, CHUNK)],
                        sg[b],
                    )
                    d.start()
                    descs.append(d)
                for d in descs:
                    d.wait()
                pltpu.make_async_copy(
                    buf, out_hbm.at[pl.ds(base + goff, GROUP)], ss[b]
                ).start()

        for b in range(NB):
            pltpu.make_async_copy(
                rows[b], out_hbm.at[pl.ds(base, GROUP)], ss[b]
            ).wait()

    return gather_kernel


_gather = _make_gather()


@jax.jit
def kernel(input_ids, weight):
    r = input_ids.astype(jnp.int32)
    rr = (r & ~(2 * PANEL - 1)) | ((r & (PANEL - 1)) << 1) | ((r // PANEL) & 1)
    ids_flat = (
        jnp.transpose(rr.T.reshape(SEQ, 2, HB), (0, 2, 1)).reshape(B_TOTAL)
    )
    table = _retile(weight.T).reshape(VOCAB_V, D)
    g = _gather(table, ids_flat)
    o_p = _outt(g.reshape(B_TOTAL // 2, 2 * D))
    return jnp.transpose(o_p, (2, 0, 1))


# speedup vs baseline: 2.1752x; 1.0522x over previous
import functools

import jax
import jax.numpy as jnp
from jax import lax
from jax.experimental import pallas as pl
from jax.experimental.pallas import tpu as pltpu
from jax.experimental.pallas import tpu_sc as plsc

VOCAB = 1000000
D = 64
B_TOTAL = 4096 * 50

PANEL = 16384
NPAN = (VOCAB + 2 * PANEL - 1) // (2 * PANEL)
VOCAB_V = NPAN * 2 * PANEL

_info = plsc.get_sparse_core_info()
NC = _info.num_cores
NS = _info.num_subcores
NW = NC * NS
B_PER_W = B_TOTAL // NW
CHUNK = 128
GC = 5
GROUP = GC * CHUNK
NG = B_PER_W // GROUP
NB = 2


def _retile_body(wt_ref, out_ref):
    x = wt_ref[...]
    out_ref[:, 0:D] = x[:, 0:PANEL].T
    out_ref[:, D:2 * D] = x[:, PANEL:2 * PANEL].T


_retile = pl.pallas_call(
    _retile_body,
    grid=(NPAN,),
    in_specs=[pl.BlockSpec((D, 2 * PANEL), lambda c: (0, c))],
    out_specs=pl.BlockSpec((PANEL, 2 * D), lambda c: (c, 0)),
    out_shape=jax.ShapeDtypeStruct((NPAN * PANEL, 2 * D), jnp.float32),
)

SEQ = 50
BATCH = 4096
HB = BATCH // 2


SB = 5


def _outt_body(g_ref, o_ref):
    x = g_ref[...]
    for e in range(SB):
        xe = x[e * HB:(e + 1) * HB, :]
        o_ref[e, :, 0:HB] = xe[:, 0:D].T
        o_ref[e, :, HB:BATCH] = xe[:, D:2 * D].T


_outt = pl.pallas_call(
    _outt_body,
    grid=(SEQ // SB,),
    in_specs=[pl.BlockSpec((SB * HB, 2 * D), lambda s: (s, 0))],
    out_specs=pl.BlockSpec((SB, D, BATCH), lambda s: (s, 0, 0)),
    out_shape=jax.ShapeDtypeStruct((SEQ, D, BATCH), jnp.float32),
)


def _make_gather():
    mesh = plsc.VectorSubcoreMesh(core_axis_name="c", subcore_axis_name="s")

    @functools.partial(
        pl.kernel,
        mesh=mesh,
        out_type=jax.ShapeDtypeStruct((B_TOTAL, D), jnp.float32),
        scratch_types=[
            pltpu.VMEM((B_PER_W,), jnp.int32),
            pltpu.VMEM((GROUP, D), jnp.float32),
            pltpu.VMEM((GROUP, D), jnp.float32),
            pltpu.SemaphoreType.DMA,
            pltpu.SemaphoreType.DMA,
            pltpu.SemaphoreType.DMA,
            pltpu.SemaphoreType.DMA,
        ],
        compiler_params=pltpu.CompilerParams(use_tc_tiling_on_sc=False),
    )
    def gather_kernel(table_hbm, ids_hbm, out_hbm,
                      idx_v, rows_a, rows_b, sg_a, sg_b, ss_a, ss_b):
        wid = lax.axis_index("s") * NC + lax.axis_index("c")
        base = wid * B_PER_W
        pltpu.sync_copy(ids_hbm.at[pl.ds(base, B_PER_W)], idx_v)

        rows = (rows_a, rows_b)
        sg = (sg_a, sg_b)
        ss = (ss_a, ss_b)

        @pl.loop(0, NG, step=NB)
        def _group(go):
            for b in range(NB):
                g = go + b
                buf = rows[b]
                @pl.when(go >= NB)
                def _():
                    pltpu.make_async_copy(
                        buf, out_hbm.at[pl.ds(base, GROUP)], ss[b]
                    ).wait()

                goff = pl.multiple_of(g * GROUP, GROUP)
                descs = []
                for c in range(GC):
                    d = pltpu.make_async_copy(
                        table_hbm.at[idx_v.at[pl.ds(goff + c * CHUNK, CHUNK)]],
                        buf.at[pl.ds(c * CHUNK, CHUNK)],
                        sg[b],
                    )
                    d.start()
                    descs.append(d)
                for d in descs:
                    d.wait()
                pltpu.make_async_copy(
                    buf, out_hbm.at[pl.ds(base + goff, GROUP)], ss[b]
                ).start()

        for b in range(NB):
            pltpu.make_async_copy(
                rows[b], out_hbm.at[pl.ds(base, GROUP)], ss[b]
            ).wait()

    return gather_kernel


_gather = _make_gather()


@jax.jit
def kernel(input_ids, weight):
    r = input_ids.astype(jnp.int32)
    rr = (r & ~(2 * PANEL - 1)) | ((r & (PANEL - 1)) << 1) | ((r // PANEL) & 1)
    ids_flat = (
        jnp.transpose(rr.T.reshape(SEQ, 2, HB), (0, 2, 1)).reshape(B_TOTAL)
    )
    table = _retile(weight.T).reshape(VOCAB_V, D)
    g = _gather(table, ids_flat)
    o_p = _outt(g.reshape(B_TOTAL // 2, 2 * D))
    return jnp.transpose(o_p, (2, 0, 1))
